# baseline (speedup 1.0000x reference)
import functools

import jax
import jax.numpy as jnp
from jax import lax
from jax.experimental import pallas as pl
from jax.experimental.pallas import tpu as pltpu
from jax.experimental.pallas import tpu_sc as plsc

N = 10000
D = 128
DA = 144
NC, NS, L = 2, 16, 16
NW = NC * NS
E_TOT = 320000 + N
E_PAD = ((E_TOT + NW * L - 1) // (NW * L)) * (NW * L)
CHUNK = E_PAD // NW
NBLK = CHUNK // L
NPAD = 10112
ROWS_PER_SUB = NPAD // NS
NSEG = 15
SEG = CHUNK // NSEG
SBLK = SEG // L
NBUF = 4

_SC_MESH = plsc.VectorSubcoreMesh(
    core_axis_name="c", subcore_axis_name="s", num_cores=NC, num_subcores=NS)


def _proj_body(x_ref, w_ref, asrc_ref, adst_ref, wl_ref, bl_ref,
               haug_ref, lin_ref, ssrc_ref, sdst_ref, mm_ref):
    i = pl.program_id(0)
    x = x_ref[...]
    h = jnp.dot(x, w_ref[...], preferred_element_type=jnp.float32)
    tail = (lax.broadcasted_iota(jnp.int32, (x.shape[0], DA - D), 1) == 0
            ).astype(jnp.float32)
    haug_ref[...] = jnp.concatenate([h, tail], axis=1)
    lin_ref[...] = jnp.dot(x, wl_ref[...],
                           preferred_element_type=jnp.float32) + bl_ref[...]
    ss = jnp.dot(h, asrc_ref[...], preferred_element_type=jnp.float32)
    sd = jnp.dot(h, adst_ref[...], preferred_element_type=jnp.float32)
    ssrc_ref[...] = ss
    sdst_ref[...] = sd
    cur = jnp.concatenate(
        [jnp.max(ss, axis=0, keepdims=True), jnp.max(sd, axis=0, keepdims=True)],
        axis=1)

    @pl.when(i == 0)
    def _():
        mm_ref[...] = cur

    @pl.when(i != 0)
    def _():
        mm_ref[...] = jnp.maximum(mm_ref[...], cur)


def _proj(x, W, a_src, a_dst, Wl, bl):
    blk = 1000
    g = N // blk
    return pl.pallas_call(
        _proj_body,
        grid=(g,),
        in_specs=[
            pl.BlockSpec((blk, D), lambda i: (i, 0)),
            pl.BlockSpec((D, D), lambda i: (0, 0)),
            pl.BlockSpec((D, 1), lambda i: (0, 0)),
            pl.BlockSpec((D, 1), lambda i: (0, 0)),
            pl.BlockSpec((D, D), lambda i: (0, 0)),
            pl.BlockSpec((1, D), lambda i: (0, 0)),
        ],
        out_specs=[
            pl.BlockSpec((blk, DA), lambda i: (i, 0)),
            pl.BlockSpec((blk, D), lambda i: (i, 0)),
            pl.BlockSpec((blk, 1), lambda i: (i, 0)),
            pl.BlockSpec((blk, 1), lambda i: (i, 0)),
            pl.BlockSpec((1, 2), lambda i: (0, 0)),
        ],
        out_shape=[
            jax.ShapeDtypeStruct((N, DA), jnp.float32),
            jax.ShapeDtypeStruct((N, D), jnp.float32),
            jax.ShapeDtypeStruct((N, 1), jnp.float32),
            jax.ShapeDtypeStruct((N, 1), jnp.float32),
            jax.ShapeDtypeStruct((1, 2), jnp.float32),
        ],
    )(x, W, a_src[:, None], a_dst[:, None], Wl, bl[None, :])


def _combine_proj_body(pa_ref, pb_ref, lin1_ref, b1_ref,
                       w_ref, asrc_ref, adst_ref, wl_ref, bl_ref,
                       haug_ref, lin_ref, ssrc_ref, sdst_ref, mm_ref):
    psum = pa_ref[...] + pb_ref[...]
    den = psum[:, D:D + 1]
    x = jnp.maximum(psum[:, :D] / den + b1_ref[...] + lin1_ref[...], 0.0)
    _proj_tail(x, w_ref, asrc_ref, adst_ref, wl_ref, bl_ref,
               haug_ref, lin_ref, ssrc_ref, sdst_ref, mm_ref)


def _proj_tail(x, w_ref, asrc_ref, adst_ref, wl_ref, bl_ref,
               haug_ref, lin_ref, ssrc_ref, sdst_ref, mm_ref):
    i = pl.program_id(0)
    h = jnp.dot(x, w_ref[...], preferred_element_type=jnp.float32)
    tail = (lax.broadcasted_iota(jnp.int32, (x.shape[0], DA - D), 1) == 0
            ).astype(jnp.float32)
    haug_ref[...] = jnp.concatenate([h, tail], axis=1)
    lin_ref[...] = jnp.dot(x, wl_ref[...],
                           preferred_element_type=jnp.float32) + bl_ref[...]
    ss = jnp.dot(h, asrc_ref[...], preferred_element_type=jnp.float32)
    sd = jnp.dot(h, adst_ref[...], preferred_element_type=jnp.float32)
    ssrc_ref[...] = ss
    sdst_ref[...] = sd
    cur = jnp.concatenate(
        [jnp.max(ss, axis=0, keepdims=True), jnp.max(sd, axis=0, keepdims=True)],
        axis=1)

    @pl.when(i == 0)
    def _():
        mm_ref[...] = cur

    @pl.when(i != 0)
    def _():
        mm_ref[...] = jnp.maximum(mm_ref[...], cur)


def _combine_proj(parts, lin1, b1, W, a_src, a_dst, Wl, bl):
    blk = 1000
    g = N // blk
    return pl.pallas_call(
        _combine_proj_body,
        grid=(g,),
        in_specs=[
            pl.BlockSpec((blk, DA), lambda i: (i, 0)),
            pl.BlockSpec((blk, DA), lambda i: (i, 0)),
            pl.BlockSpec((blk, D), lambda i: (i, 0)),
            pl.BlockSpec((1, D), lambda i: (0, 0)),
            pl.BlockSpec((D, D), lambda i: (0, 0)),
            pl.BlockSpec((D, 1), lambda i: (0, 0)),
            pl.BlockSpec((D, 1), lambda i: (0, 0)),
            pl.BlockSpec((D, D), lambda i: (0, 0)),
            pl.BlockSpec((1, D), lambda i: (0, 0)),
        ],
        out_specs=[
            pl.BlockSpec((blk, DA), lambda i: (i, 0)),
            pl.BlockSpec((blk, D), lambda i: (i, 0)),
            pl.BlockSpec((blk, 1), lambda i: (i, 0)),
            pl.BlockSpec((blk, 1), lambda i: (i, 0)),
            pl.BlockSpec((1, 2), lambda i: (0, 0)),
        ],
        out_shape=[
            jax.ShapeDtypeStruct((N, DA), jnp.float32),
            jax.ShapeDtypeStruct((N, D), jnp.float32),
            jax.ShapeDtypeStruct((N, 1), jnp.float32),
            jax.ShapeDtypeStruct((N, 1), jnp.float32),
            jax.ShapeDtypeStruct((1, 2), jnp.float32),
        ],
    )(parts[0], parts[1], lin1, b1[None, :], W, a_src[:, None], a_dst[:, None],
      Wl, bl[None, :])


def _agg_body(src_h, dst_h, ssrc_h, sdst_h, mvec_h, haug_h, zeros_h,
              parts_h,
              ssrc_v, sdst_v, m_v, src_v, dst_v,
              gb0, gb1, gb2, gb3,
              sb0, sb1, sb2, sb3,
              acc_sh,
              gsem0, gsem1, gsem2, gsem3,
              ssem0, ssem1, ssem2, ssem3):
    gbufs = (gb0, gb1, gb2, gb3)
    sbufs = (sb0, sb1, sb2, sb3)
    gsems = (gsem0, gsem1, gsem2, gsem3)
    ssems = (ssem0, ssem1, ssem2, ssem3)
    cid = lax.axis_index("c")
    sid = lax.axis_index("s")
    wid = cid * NS + sid

    pltpu.sync_copy(ssrc_h, ssrc_v)
    pltpu.sync_copy(sdst_h, sdst_v)
    pltpu.sync_copy(mvec_h, m_v)
    mvec = m_v[...]

    row0 = sid * ROWS_PER_SUB
    pltpu.sync_copy(zeros_h.at[pl.ds(row0, ROWS_PER_SUB)],
                    acc_sh.at[pl.ds(row0, ROWS_PER_SUB)])
    plsc.subcore_barrier()

    lane = lax.iota(jnp.int32, L)

    def seg(si, _):
        off = wid * CHUNK + si * SEG
        pltpu.sync_copy(src_h.at[pl.ds(off, SEG)], src_v)
        pltpu.sync_copy(dst_h.at[pl.ds(off, SEG)], dst_v)

        for b in range(NBUF):
            svp = src_v[pl.ds(b * L, L)]
            pltpu.async_copy(haug_h.at[svp], gbufs[b], gsems[b])

        def use(g, b, issue_next, wait_scatter):
            sv = src_v[pl.ds(g * L, L)]
            dv = dst_v[pl.ds(g * L, L)]
            a = plsc.load_gather(ssrc_v, [sv])
            bsc = plsc.load_gather(sdst_v, [dv])
            t = a + bsc
            e = jnp.maximum(t, 0.2 * t)
            p = jnp.exp(e - mvec)
            gidx = off + g * L + lane
            p = jnp.where(gidx < E_TOT, p, 0.0)
            pltpu.make_async_copy(haug_h.at[sv], gbufs[b], gsems[b]).wait()
            if wait_scatter:
                pltpu.make_async_copy(sbufs[b], acc_sh.at[dv], ssems[b]).wait()
            for j in range(L):
                ps = p[j]
                for c in range(DA // L):
                    sl = pl.ds(c * L, L)
                    sbufs[b][j, sl] = gbufs[b][j, sl] * ps
            if issue_next is True:
                svn = src_v[pl.ds((g + NBUF) * L, L)]
                pltpu.async_copy(haug_h.at[svn], gbufs[b], gsems[b])
            elif issue_next is not False:
                @pl.when(issue_next)
                def _():
                    svn = src_v[pl.ds((g + NBUF) * L, L)]
                    pltpu.async_copy(haug_h.at[svn], gbufs[b], gsems[b])
            pltpu.async_copy(sbufs[b], acc_sh.at[dv], ssems[b], add=True)

        for b in range(NBUF):
            use(b, b, NBUF + b < SBLK, False)

        def group(gi, _):
            for b in range(NBUF):
                g = gi * NBUF + b
                use(g, b, g + NBUF < SBLK, True)
            return ()

        lax.fori_loop(1, SBLK // NBUF, group, (), unroll=False)
        for g in range((SBLK // NBUF) * NBUF, SBLK):
            use(g, g % NBUF, False, True)
        dv0 = dst_v[pl.ds(0, L)]
        for b in range(NBUF):
            pltpu.make_async_copy(sbufs[b], acc_sh.at[dv0], ssems[b]).wait()
        return ()

    lax.fori_loop(0, NSEG, seg, (), unroll=False)
    plsc.subcore_barrier()
    pltpu.sync_copy(acc_sh.at[pl.ds(row0, ROWS_PER_SUB)],
                    parts_h.at[cid].at[pl.ds(row0, ROWS_PER_SUB)])


_agg = functools.partial(
    pl.kernel,
    out_type=jax.ShapeDtypeStruct((NC, NPAD, DA), jnp.float32),
    mesh=_SC_MESH,
    compiler_params=pltpu.CompilerParams(needs_layout_passes=False, use_tc_tiling_on_sc=False),
    scratch_types=(
        [pltpu.VMEM((N,), jnp.float32),
         pltpu.VMEM((N,), jnp.float32),
         pltpu.VMEM((L,), jnp.float32),
         pltpu.VMEM((SEG,), jnp.int32),
         pltpu.VMEM((SEG,), jnp.int32)]
        + [pltpu.VMEM((L, DA), jnp.float32)] * (2 * NBUF)
        + [pltpu.VMEM_SHARED((NPAD, DA), jnp.float32)]
        + [pltpu.SemaphoreType.DMA] * (2 * NBUF)
    ),
)(_agg_body)


PPW = 2048 // NW
QPW = 256 // NW


def _gather_body(pa_h, pb_h, lin_h, b2_h, pids_h, xproj_h, prids_h,
                 pemb_h, xprojg_h,
                 pid_v, prid_v, ra_v, rb_v, linr_v, b2_v, out_v, xpr_v, sem):
    cid = lax.axis_index("c")
    sid = lax.axis_index("s")
    wid = cid * NS + sid

    pltpu.sync_copy(b2_h, b2_v)
    pltpu.sync_copy(pids_h.at[pl.ds(wid * PPW, PPW)], pid_v)
    pltpu.async_copy(pa_h.at[pid_v], ra_v, sem).wait()
    pltpu.async_copy(pb_h.at[pid_v], rb_v, sem).wait()
    pltpu.async_copy(lin_h.at[pid_v], linr_v, sem).wait()

    def row(j, _):
        dena = ra_v[j, pl.ds(D, DA - D)]
        denb = rb_v[j, pl.ds(D, DA - D)]
        invv = 1.0 / (dena + denb)
        inv = invv[0]
        for c in range(D // L):
            sl = pl.ds(c * L, L)
            out_v[j, sl] = ((ra_v[j, sl] + rb_v[j, sl]) * inv
                            + b2_v[pl.ds(c * L, L)] + linr_v[j, sl])
        return ()

    lax.fori_loop(0, PPW, row, (), unroll=False)
    pltpu.sync_copy(out_v, pemb_h.at[pl.ds(wid * PPW, PPW)])

    pltpu.sync_copy(prids_h.at[pl.ds(wid * QPW, QPW)], prid_v)
    pltpu.async_copy(xproj_h.at[prid_v], xpr_v, sem).wait()
    pltpu.sync_copy(xpr_v, xprojg_h.at[pl.ds(wid * QPW, QPW)])


def _gather_make(nproj, dproj):
    return functools.partial(
        pl.kernel,
        out_type=[
            jax.ShapeDtypeStruct((2048, D), jnp.float32),
            jax.ShapeDtypeStruct((256, dproj), jnp.float32),
        ],
        mesh=_SC_MESH,
        compiler_params=pltpu.CompilerParams(needs_layout_passes=False, use_tc_tiling_on_sc=False),
        scratch_types=[
            pltpu.VMEM((PPW,), jnp.int32),
            pltpu.VMEM((QPW,), jnp.int32),
            pltpu.VMEM((PPW, DA), jnp.float32),
            pltpu.VMEM((PPW, DA), jnp.float32),
            pltpu.VMEM((PPW, D), jnp.float32),
            pltpu.VMEM((D,), jnp.float32),
            pltpu.VMEM((PPW, D), jnp.float32),
            pltpu.VMEM((QPW, dproj), jnp.float32),
            pltpu.SemaphoreType.DMA,
        ],
    )(_gather_body)


def _head_a_body(xp_ref, wfc3_ref, bfc3_ref, wbot_ref, bfc1_ref, out_ref):
    pr = jnp.dot(xp_ref[...], wfc3_ref[...],
                 preferred_element_type=jnp.float32) + bfc3_ref[...]
    out_ref[...] = jnp.dot(pr, wbot_ref[...],
                           preferred_element_type=jnp.float32) + bfc1_ref[...]


def _head_b_body(pe_ref, wtop_ref, rep_ref, wfc2_ref, out_ref):
    mix = jnp.maximum(
        jnp.dot(pe_ref[...], wtop_ref[...], preferred_element_type=jnp.float32)
        + rep_ref[...], 0.0)
    out_ref[...] = jnp.dot(mix, wfc2_ref[...],
                           preferred_element_type=jnp.float32)


def kernel(x_person, edge_index, x_project, person_ids, project_ids, W1, a_src1, a_dst1, b1, Wl1, bl1, W2, a_src2, a_dst2, b2, Wl2, bl2, Wfc1, bfc1, Wfc2, bfc2, Wfc3, bfc3):
    loops = jnp.arange(N, dtype=edge_index.dtype)
    src = jnp.concatenate([edge_index[0], loops])
    dst = jnp.concatenate([edge_index[1], loops])
    pad = E_PAD - E_TOT
    src = jnp.pad(src, (0, pad))
    dst = jnp.pad(dst, (0, pad))
    zeros = jnp.zeros((NPAD, DA), jnp.float32)

    def mbound(mm):
        t = mm[0, 0] + mm[0, 1]
        return jnp.full((L,), jnp.maximum(t, 0.2 * t), jnp.float32)

    haug1, lin1, ss1, sd1, mm1 = _proj(x_person, W1, a_src1, a_dst1, Wl1, bl1)
    parts1 = _agg(src, dst, ss1[:, 0], sd1[:, 0], mbound(mm1), haug1, zeros)

    haug2, lin2, ss2, sd2, mm2 = _combine_proj(
        parts1[:, :N], lin1, b1, W2, a_src2, a_dst2, Wl2, bl2)
    parts2 = _agg(src, dst, ss2[:, 0], sd2[:, 0], mbound(mm2), haug2, zeros)

    pemb, xprojg = _gather_make(x_project.shape[0], x_project.shape[1])(
        parts2[0], parts2[1], lin2, b2, person_ids, x_project, project_ids)

    pr1 = pl.pallas_call(
        _head_a_body,
        out_shape=jax.ShapeDtypeStruct((256, D), jnp.float32),
    )(xprojg, Wfc3, bfc3[None, :], Wfc1[D:], bfc1[None, :])
    rep = jnp.repeat(pr1, 2048 // 256, axis=0)
    out = pl.pallas_call(
        _head_b_body,
        out_shape=jax.ShapeDtypeStruct((2048, 1), jnp.float32),
    )(pemb, Wfc1[:D], rep, Wfc2)
    return out.reshape(256, 2048 // 256) + bfc2[0]


# speedup vs baseline: 1.0060x; 1.0060x over previous
import functools

import jax
import jax.numpy as jnp
from jax import lax
from jax.experimental import pallas as pl
from jax.experimental.pallas import tpu as pltpu
from jax.experimental.pallas import tpu_sc as plsc

N = 10000
D = 128
DA = 144
NC, NS, L = 2, 16, 16
NW = NC * NS
E_TOT = 320000 + N
E_PAD = ((E_TOT + NW * L - 1) // (NW * L)) * (NW * L)
CHUNK = E_PAD // NW
NBLK = CHUNK // L
NPAD = 10112
ROWS_PER_SUB = NPAD // NS
NSEG = 15
SEG = CHUNK // NSEG
SBLK = SEG // L
NBUF = 4

_SC_MESH = plsc.VectorSubcoreMesh(
    core_axis_name="c", subcore_axis_name="s", num_cores=NC, num_subcores=NS)


def _proj_body(x_ref, w_ref, asrc_ref, adst_ref, wl_ref, bl_ref,
               haug_ref, lin_ref, ssrc_ref, sdst_ref, mm_ref):
    i = pl.program_id(0)
    x = x_ref[...]
    h = jnp.dot(x, w_ref[...], preferred_element_type=jnp.float32)
    tail = (lax.broadcasted_iota(jnp.int32, (x.shape[0], DA - D), 1) == 0
            ).astype(jnp.float32)
    haug_ref[...] = jnp.concatenate([h, tail], axis=1)
    lin_ref[...] = jnp.dot(x, wl_ref[...],
                           preferred_element_type=jnp.float32) + bl_ref[...]
    ss = jnp.dot(h, asrc_ref[...], preferred_element_type=jnp.float32)
    sd = jnp.dot(h, adst_ref[...], preferred_element_type=jnp.float32)
    ssrc_ref[...] = ss
    sdst_ref[...] = sd
    cur = jnp.concatenate(
        [jnp.max(ss, axis=0, keepdims=True), jnp.max(sd, axis=0, keepdims=True)],
        axis=1)

    @pl.when(i == 0)
    def _():
        mm_ref[...] = cur

    @pl.when(i != 0)
    def _():
        mm_ref[...] = jnp.maximum(mm_ref[...], cur)


def _proj(x, W, a_src, a_dst, Wl, bl):
    blk = 1000
    g = N // blk
    return pl.pallas_call(
        _proj_body,
        grid=(g,),
        in_specs=[
            pl.BlockSpec((blk, D), lambda i: (i, 0)),
            pl.BlockSpec((D, D), lambda i: (0, 0)),
            pl.BlockSpec((D, 1), lambda i: (0, 0)),
            pl.BlockSpec((D, 1), lambda i: (0, 0)),
            pl.BlockSpec((D, D), lambda i: (0, 0)),
            pl.BlockSpec((1, D), lambda i: (0, 0)),
        ],
        out_specs=[
            pl.BlockSpec((blk, DA), lambda i: (i, 0)),
            pl.BlockSpec((blk, D), lambda i: (i, 0)),
            pl.BlockSpec((blk, 1), lambda i: (i, 0)),
            pl.BlockSpec((blk, 1), lambda i: (i, 0)),
            pl.BlockSpec((1, 2), lambda i: (0, 0)),
        ],
        out_shape=[
            jax.ShapeDtypeStruct((N, DA), jnp.float32),
            jax.ShapeDtypeStruct((N, D), jnp.float32),
            jax.ShapeDtypeStruct((N, 1), jnp.float32),
            jax.ShapeDtypeStruct((N, 1), jnp.float32),
            jax.ShapeDtypeStruct((1, 2), jnp.float32),
        ],
    )(x, W, a_src[:, None], a_dst[:, None], Wl, bl[None, :])


def _combine_proj_body(pa_ref, pb_ref, lin1_ref, b1_ref,
                       w_ref, asrc_ref, adst_ref, wl_ref, bl_ref,
                       haug_ref, lin_ref, ssrc_ref, sdst_ref, mm_ref):
    psum = pa_ref[...] + pb_ref[...]
    den = psum[:, D:D + 1]
    x = jnp.maximum(psum[:, :D] / den + b1_ref[...] + lin1_ref[...], 0.0)
    _proj_tail(x, w_ref, asrc_ref, adst_ref, wl_ref, bl_ref,
               haug_ref, lin_ref, ssrc_ref, sdst_ref, mm_ref)


def _proj_tail(x, w_ref, asrc_ref, adst_ref, wl_ref, bl_ref,
               haug_ref, lin_ref, ssrc_ref, sdst_ref, mm_ref):
    i = pl.program_id(0)
    h = jnp.dot(x, w_ref[...], preferred_element_type=jnp.float32)
    tail = (lax.broadcasted_iota(jnp.int32, (x.shape[0], DA - D), 1) == 0
            ).astype(jnp.float32)
    haug_ref[...] = jnp.concatenate([h, tail], axis=1)
    lin_ref[...] = jnp.dot(x, wl_ref[...],
                           preferred_element_type=jnp.float32) + bl_ref[...]
    ss = jnp.dot(h, asrc_ref[...], preferred_element_type=jnp.float32)
    sd = jnp.dot(h, adst_ref[...], preferred_element_type=jnp.float32)
    ssrc_ref[...] = ss
    sdst_ref[...] = sd
    cur = jnp.concatenate(
        [jnp.max(ss, axis=0, keepdims=True), jnp.max(sd, axis=0, keepdims=True)],
        axis=1)

    @pl.when(i == 0)
    def _():
        mm_ref[...] = cur

    @pl.when(i != 0)
    def _():
        mm_ref[...] = jnp.maximum(mm_ref[...], cur)


def _combine_proj(parts, lin1, b1, W, a_src, a_dst, Wl, bl):
    blk = 1000
    g = N // blk
    return pl.pallas_call(
        _combine_proj_body,
        grid=(g,),
        in_specs=[
            pl.BlockSpec((blk, DA), lambda i: (i, 0)),
            pl.BlockSpec((blk, DA), lambda i: (i, 0)),
            pl.BlockSpec((blk, D), lambda i: (i, 0)),
            pl.BlockSpec((1, D), lambda i: (0, 0)),
            pl.BlockSpec((D, D), lambda i: (0, 0)),
            pl.BlockSpec((D, 1), lambda i: (0, 0)),
            pl.BlockSpec((D, 1), lambda i: (0, 0)),
            pl.BlockSpec((D, D), lambda i: (0, 0)),
            pl.BlockSpec((1, D), lambda i: (0, 0)),
        ],
        out_specs=[
            pl.BlockSpec((blk, DA), lambda i: (i, 0)),
            pl.BlockSpec((blk, D), lambda i: (i, 0)),
            pl.BlockSpec((blk, 1), lambda i: (i, 0)),
            pl.BlockSpec((blk, 1), lambda i: (i, 0)),
            pl.BlockSpec((1, 2), lambda i: (0, 0)),
        ],
        out_shape=[
            jax.ShapeDtypeStruct((N, DA), jnp.float32),
            jax.ShapeDtypeStruct((N, D), jnp.float32),
            jax.ShapeDtypeStruct((N, 1), jnp.float32),
            jax.ShapeDtypeStruct((N, 1), jnp.float32),
            jax.ShapeDtypeStruct((1, 2), jnp.float32),
        ],
    )(parts[0], parts[1], lin1, b1[None, :], W, a_src[:, None], a_dst[:, None],
      Wl, bl[None, :])


def _agg_body(src_h, dst_h, ssrc_h, sdst_h, mvec_h, haug_h, zeros_h,
              parts_h,
              ssrc_v, sdst_v, m_v, src_v, dst_v,
              gb0, gb1, gb2, gb3,
              sb0, sb1, sb2, sb3,
              acc_sh,
              gsem0, gsem1, gsem2, gsem3,
              ssem0, ssem1, ssem2, ssem3):
    gbufs = (gb0, gb1, gb2, gb3)
    sbufs = (sb0, sb1, sb2, sb3)
    gsems = (gsem0, gsem1, gsem2, gsem3)
    ssems = (ssem0, ssem1, ssem2, ssem3)
    cid = lax.axis_index("c")
    sid = lax.axis_index("s")
    wid = cid * NS + sid

    pltpu.sync_copy(ssrc_h, ssrc_v)
    pltpu.sync_copy(sdst_h, sdst_v)
    pltpu.sync_copy(mvec_h, m_v)
    mvec = m_v[...]

    row0 = sid * ROWS_PER_SUB
    pltpu.sync_copy(zeros_h.at[pl.ds(row0, ROWS_PER_SUB)],
                    acc_sh.at[pl.ds(row0, ROWS_PER_SUB)])
    plsc.subcore_barrier()

    lane = lax.iota(jnp.int32, L)

    def seg(si, _):
        off = wid * CHUNK + si * SEG
        pltpu.sync_copy(src_h.at[pl.ds(off, SEG)], src_v)
        pltpu.sync_copy(dst_h.at[pl.ds(off, SEG)], dst_v)

        for b in range(NBUF):
            svp = src_v[pl.ds(b * L, L)]
            pltpu.async_copy(haug_h.at[svp], gbufs[b], gsems[b])

        def use(g, b, issue_next, wait_scatter):
            sv = src_v[pl.ds(g * L, L)]
            dv = dst_v[pl.ds(g * L, L)]
            a = plsc.load_gather(ssrc_v, [sv])
            bsc = plsc.load_gather(sdst_v, [dv])
            t = a + bsc
            e = jnp.maximum(t, 0.2 * t)
            p = jnp.exp(e - mvec)
            gidx = off + g * L + lane
            p = jnp.where(gidx < E_TOT, p, 0.0)
            pltpu.make_async_copy(haug_h.at[sv], gbufs[b], gsems[b]).wait()
            for j in range(L):
                ps = p[j]
                for c in range(DA // L):
                    sl = pl.ds(c * L, L)
                    sbufs[b][j, sl] = gbufs[b][j, sl] * ps
            if issue_next is True:
                svn = src_v[pl.ds((g + NBUF) * L, L)]
                pltpu.async_copy(haug_h.at[svn], gbufs[b], gsems[b])
            elif issue_next is not False:
                @pl.when(issue_next)
                def _():
                    svn = src_v[pl.ds((g + NBUF) * L, L)]
                    pltpu.async_copy(haug_h.at[svn], gbufs[b], gsems[b])
            if wait_scatter:
                bp = (b - 1) % NBUF
                pltpu.make_async_copy(sbufs[bp], acc_sh.at[dv],
                                      ssems[0]).wait()
            pltpu.async_copy(sbufs[b], acc_sh.at[dv], ssems[0], add=True)

        for b in range(NBUF):
            use(b, b, NBUF + b < SBLK, b > 0)

        def group(gi, _):
            for b in range(NBUF):
                g = gi * NBUF + b
                use(g, b, g + NBUF < SBLK, True)
            return ()

        lax.fori_loop(1, SBLK // NBUF, group, (), unroll=False)
        for g in range((SBLK // NBUF) * NBUF, SBLK):
            use(g, g % NBUF, False, True)
        dv0 = dst_v[pl.ds(0, L)]
        pltpu.make_async_copy(sbufs[(SBLK - 1) % NBUF], acc_sh.at[dv0],
                              ssems[0]).wait()
        return ()

    lax.fori_loop(0, NSEG, seg, (), unroll=False)
    plsc.subcore_barrier()
    pltpu.sync_copy(acc_sh.at[pl.ds(row0, ROWS_PER_SUB)],
                    parts_h.at[cid].at[pl.ds(row0, ROWS_PER_SUB)])


_agg = functools.partial(
    pl.kernel,
    out_type=jax.ShapeDtypeStruct((NC, NPAD, DA), jnp.float32),
    mesh=_SC_MESH,
    compiler_params=pltpu.CompilerParams(needs_layout_passes=False, use_tc_tiling_on_sc=False),
    scratch_types=(
        [pltpu.VMEM((N,), jnp.float32),
         pltpu.VMEM((N,), jnp.float32),
         pltpu.VMEM((L,), jnp.float32),
         pltpu.VMEM((SEG,), jnp.int32),
         pltpu.VMEM((SEG,), jnp.int32)]
        + [pltpu.VMEM((L, DA), jnp.float32)] * (2 * NBUF)
        + [pltpu.VMEM_SHARED((NPAD, DA), jnp.float32)]
        + [pltpu.SemaphoreType.DMA] * (2 * NBUF)
    ),
)(_agg_body)


PPW = 2048 // NW
QPW = 256 // NW


def _gather_body(pa_h, pb_h, lin_h, b2_h, pids_h, xproj_h, prids_h,
                 pemb_h, xprojg_h,
                 pid_v, prid_v, ra_v, rb_v, linr_v, b2_v, out_v, xpr_v, sem):
    cid = lax.axis_index("c")
    sid = lax.axis_index("s")
    wid = cid * NS + sid

    pltpu.sync_copy(b2_h, b2_v)
    pltpu.sync_copy(pids_h.at[pl.ds(wid * PPW, PPW)], pid_v)
    pltpu.async_copy(pa_h.at[pid_v], ra_v, sem).wait()
    pltpu.async_copy(pb_h.at[pid_v], rb_v, sem).wait()
    pltpu.async_copy(lin_h.at[pid_v], linr_v, sem).wait()

    def row(j, _):
        dena = ra_v[j, pl.ds(D, DA - D)]
        denb = rb_v[j, pl.ds(D, DA - D)]
        invv = 1.0 / (dena + denb)
        inv = invv[0]
        for c in range(D // L):
            sl = pl.ds(c * L, L)
            out_v[j, sl] = ((ra_v[j, sl] + rb_v[j, sl]) * inv
                            + b2_v[pl.ds(c * L, L)] + linr_v[j, sl])
        return ()

    lax.fori_loop(0, PPW, row, (), unroll=False)
    pltpu.sync_copy(out_v, pemb_h.at[pl.ds(wid * PPW, PPW)])

    pltpu.sync_copy(prids_h.at[pl.ds(wid * QPW, QPW)], prid_v)
    pltpu.async_copy(xproj_h.at[prid_v], xpr_v, sem).wait()
    pltpu.sync_copy(xpr_v, xprojg_h.at[pl.ds(wid * QPW, QPW)])


def _gather_make(nproj, dproj):
    return functools.partial(
        pl.kernel,
        out_type=[
            jax.ShapeDtypeStruct((2048, D), jnp.float32),
            jax.ShapeDtypeStruct((256, dproj), jnp.float32),
        ],
        mesh=_SC_MESH,
        compiler_params=pltpu.CompilerParams(needs_layout_passes=False, use_tc_tiling_on_sc=False),
        scratch_types=[
            pltpu.VMEM((PPW,), jnp.int32),
            pltpu.VMEM((QPW,), jnp.int32),
            pltpu.VMEM((PPW, DA), jnp.float32),
            pltpu.VMEM((PPW, DA), jnp.float32),
            pltpu.VMEM((PPW, D), jnp.float32),
            pltpu.VMEM((D,), jnp.float32),
            pltpu.VMEM((PPW, D), jnp.float32),
            pltpu.VMEM((QPW, dproj), jnp.float32),
            pltpu.SemaphoreType.DMA,
        ],
    )(_gather_body)


def _head_a_body(xp_ref, wfc3_ref, bfc3_ref, wbot_ref, bfc1_ref, out_ref):
    pr = jnp.dot(xp_ref[...], wfc3_ref[...],
                 preferred_element_type=jnp.float32) + bfc3_ref[...]
    out_ref[...] = jnp.dot(pr, wbot_ref[...],
                           preferred_element_type=jnp.float32) + bfc1_ref[...]


def _head_b_body(pe_ref, wtop_ref, rep_ref, wfc2_ref, out_ref):
    mix = jnp.maximum(
        jnp.dot(pe_ref[...], wtop_ref[...], preferred_element_type=jnp.float32)
        + rep_ref[...], 0.0)
    out_ref[...] = jnp.dot(mix, wfc2_ref[...],
                           preferred_element_type=jnp.float32)


def kernel(x_person, edge_index, x_project, person_ids, project_ids, W1, a_src1, a_dst1, b1, Wl1, bl1, W2, a_src2, a_dst2, b2, Wl2, bl2, Wfc1, bfc1, Wfc2, bfc2, Wfc3, bfc3):
    loops = jnp.arange(N, dtype=edge_index.dtype)
    src = jnp.concatenate([edge_index[0], loops])
    dst = jnp.concatenate([edge_index[1], loops])
    pad = E_PAD - E_TOT
    src = jnp.pad(src, (0, pad))
    dst = jnp.pad(dst, (0, pad))
    zeros = jnp.zeros((NPAD, DA), jnp.float32)

    def mbound(mm):
        t = mm[0, 0] + mm[0, 1]
        return jnp.full((L,), jnp.maximum(t, 0.2 * t), jnp.float32)

    haug1, lin1, ss1, sd1, mm1 = _proj(x_person, W1, a_src1, a_dst1, Wl1, bl1)
    parts1 = _agg(src, dst, ss1[:, 0], sd1[:, 0], mbound(mm1), haug1, zeros)

    haug2, lin2, ss2, sd2, mm2 = _combine_proj(
        parts1[:, :N], lin1, b1, W2, a_src2, a_dst2, Wl2, bl2)
    parts2 = _agg(src, dst, ss2[:, 0], sd2[:, 0], mbound(mm2), haug2, zeros)

    pemb, xprojg = _gather_make(x_project.shape[0], x_project.shape[1])(
        parts2[0], parts2[1], lin2, b2, person_ids, x_project, project_ids)

    pr1 = pl.pallas_call(
        _head_a_body,
        out_shape=jax.ShapeDtypeStruct((256, D), jnp.float32),
    )(xprojg, Wfc3, bfc3[None, :], Wfc1[D:], bfc1[None, :])
    rep = jnp.repeat(pr1, 2048 // 256, axis=0)
    out = pl.pallas_call(
        _head_b_body,
        out_shape=jax.ShapeDtypeStruct((2048, 1), jnp.float32),
    )(pemb, Wfc1[:D], rep, Wfc2)
    return out.reshape(256, 2048 // 256) + bfc2[0]


# speedup vs baseline: 1.0120x; 1.0060x over previous
import functools

import jax
import jax.numpy as jnp
from jax import lax
from jax.experimental import pallas as pl
from jax.experimental.pallas import tpu as pltpu
from jax.experimental.pallas import tpu_sc as plsc

N = 10000
D = 128
DA = 144
NC, NS, L = 2, 16, 16
NW = NC * NS
E_TOT = 320000 + N
E_PAD = ((E_TOT + NW * L - 1) // (NW * L)) * (NW * L)
CHUNK = E_PAD // NW
NBLK = CHUNK // L
NPAD = 10112
ROWS_PER_SUB = NPAD // NS
NSEG = 15
SEG = CHUNK // NSEG
SBLK = SEG // L
NBUF = 4

_SC_MESH = plsc.VectorSubcoreMesh(
    core_axis_name="c", subcore_axis_name="s", num_cores=NC, num_subcores=NS)


def _proj_body(x_ref, w_ref, asrc_ref, adst_ref, wl_ref, bl_ref,
               haug_ref, lin_ref, ssrc_ref, sdst_ref, mm_ref):
    i = pl.program_id(0)
    x = x_ref[...]
    h = jnp.dot(x, w_ref[...], preferred_element_type=jnp.float32)
    tail = (lax.broadcasted_iota(jnp.int32, (x.shape[0], DA - D), 1) == 0
            ).astype(jnp.float32)
    haug_ref[...] = jnp.concatenate([h, tail], axis=1)
    lin_ref[...] = jnp.dot(x, wl_ref[...],
                           preferred_element_type=jnp.float32) + bl_ref[...]
    ss = jnp.dot(h, asrc_ref[...], preferred_element_type=jnp.float32)
    sd = jnp.dot(h, adst_ref[...], preferred_element_type=jnp.float32)
    ssrc_ref[...] = ss
    sdst_ref[...] = sd
    cur = jnp.concatenate(
        [jnp.max(ss, axis=0, keepdims=True), jnp.max(sd, axis=0, keepdims=True)],
        axis=1)

    @pl.when(i == 0)
    def _():
        mm_ref[...] = cur

    @pl.when(i != 0)
    def _():
        mm_ref[...] = jnp.maximum(mm_ref[...], cur)


def _proj(x, W, a_src, a_dst, Wl, bl):
    blk = 1000
    g = N // blk
    return pl.pallas_call(
        _proj_body,
        grid=(g,),
        in_specs=[
            pl.BlockSpec((blk, D), lambda i: (i, 0)),
            pl.BlockSpec((D, D), lambda i: (0, 0)),
            pl.BlockSpec((D, 1), lambda i: (0, 0)),
            pl.BlockSpec((D, 1), lambda i: (0, 0)),
            pl.BlockSpec((D, D), lambda i: (0, 0)),
            pl.BlockSpec((1, D), lambda i: (0, 0)),
        ],
        out_specs=[
            pl.BlockSpec((blk, DA), lambda i: (i, 0)),
            pl.BlockSpec((blk, D), lambda i: (i, 0)),
            pl.BlockSpec((blk, 1), lambda i: (i, 0)),
            pl.BlockSpec((blk, 1), lambda i: (i, 0)),
            pl.BlockSpec((1, 2), lambda i: (0, 0)),
        ],
        out_shape=[
            jax.ShapeDtypeStruct((N, DA), jnp.float32),
            jax.ShapeDtypeStruct((N, D), jnp.float32),
            jax.ShapeDtypeStruct((N, 1), jnp.float32),
            jax.ShapeDtypeStruct((N, 1), jnp.float32),
            jax.ShapeDtypeStruct((1, 2), jnp.float32),
        ],
    )(x, W, a_src[:, None], a_dst[:, None], Wl, bl[None, :])


def _combine_proj_body(pa_ref, pb_ref, lin1_ref, b1_ref,
                       w_ref, asrc_ref, adst_ref, wl_ref, bl_ref,
                       haug_ref, lin_ref, ssrc_ref, sdst_ref, mm_ref):
    psum = pa_ref[...] + pb_ref[...]
    den = psum[:, D:D + 1]
    x = jnp.maximum(psum[:, :D] / den + b1_ref[...] + lin1_ref[...], 0.0)
    _proj_tail(x, w_ref, asrc_ref, adst_ref, wl_ref, bl_ref,
               haug_ref, lin_ref, ssrc_ref, sdst_ref, mm_ref)


def _proj_tail(x, w_ref, asrc_ref, adst_ref, wl_ref, bl_ref,
               haug_ref, lin_ref, ssrc_ref, sdst_ref, mm_ref):
    i = pl.program_id(0)
    h = jnp.dot(x, w_ref[...], preferred_element_type=jnp.float32)
    tail = (lax.broadcasted_iota(jnp.int32, (x.shape[0], DA - D), 1) == 0
            ).astype(jnp.float32)
    haug_ref[...] = jnp.concatenate([h, tail], axis=1)
    lin_ref[...] = jnp.dot(x, wl_ref[...],
                           preferred_element_type=jnp.float32) + bl_ref[...]
    ss = jnp.dot(h, asrc_ref[...], preferred_element_type=jnp.float32)
    sd = jnp.dot(h, adst_ref[...], preferred_element_type=jnp.float32)
    ssrc_ref[...] = ss
    sdst_ref[...] = sd
    cur = jnp.concatenate(
        [jnp.max(ss, axis=0, keepdims=True), jnp.max(sd, axis=0, keepdims=True)],
        axis=1)

    @pl.when(i == 0)
    def _():
        mm_ref[...] = cur

    @pl.when(i != 0)
    def _():
        mm_ref[...] = jnp.maximum(mm_ref[...], cur)


def _combine_proj(parts, lin1, b1, W, a_src, a_dst, Wl, bl):
    blk = 1000
    g = N // blk
    return pl.pallas_call(
        _combine_proj_body,
        grid=(g,),
        in_specs=[
            pl.BlockSpec((blk, DA), lambda i: (i, 0)),
            pl.BlockSpec((blk, DA), lambda i: (i, 0)),
            pl.BlockSpec((blk, D), lambda i: (i, 0)),
            pl.BlockSpec((1, D), lambda i: (0, 0)),
            pl.BlockSpec((D, D), lambda i: (0, 0)),
            pl.BlockSpec((D, 1), lambda i: (0, 0)),
            pl.BlockSpec((D, 1), lambda i: (0, 0)),
            pl.BlockSpec((D, D), lambda i: (0, 0)),
            pl.BlockSpec((1, D), lambda i: (0, 0)),
        ],
        out_specs=[
            pl.BlockSpec((blk, DA), lambda i: (i, 0)),
            pl.BlockSpec((blk, D), lambda i: (i, 0)),
            pl.BlockSpec((blk, 1), lambda i: (i, 0)),
            pl.BlockSpec((blk, 1), lambda i: (i, 0)),
            pl.BlockSpec((1, 2), lambda i: (0, 0)),
        ],
        out_shape=[
            jax.ShapeDtypeStruct((N, DA), jnp.float32),
            jax.ShapeDtypeStruct((N, D), jnp.float32),
            jax.ShapeDtypeStruct((N, 1), jnp.float32),
            jax.ShapeDtypeStruct((N, 1), jnp.float32),
            jax.ShapeDtypeStruct((1, 2), jnp.float32),
        ],
    )(parts[0], parts[1], lin1, b1[None, :], W, a_src[:, None], a_dst[:, None],
      Wl, bl[None, :])


def _agg_body(src_h, dst_h, ssrc_h, sdst_h, mvec_h, haug_h, zeros_h,
              parts_h,
              ssrc_v, sdst_v, m_v, src_v, dst_v,
              gb0, gb1, gb2, gb3,
              sb0, sb1, sb2, sb3,
              acc_sh,
              gsem0, gsem1, gsem2, gsem3,
              ssem0, ssem1, ssem2, ssem3):
    gbufs = (gb0, gb1, gb2, gb3)
    sbufs = (sb0, sb1, sb2, sb3)
    gsems = (gsem0, gsem1, gsem2, gsem3)
    ssems = (ssem0, ssem1, ssem2, ssem3)
    cid = lax.axis_index("c")
    sid = lax.axis_index("s")
    wid = cid * NS + sid

    pltpu.sync_copy(ssrc_h, ssrc_v)
    pltpu.sync_copy(sdst_h, sdst_v)
    pltpu.sync_copy(mvec_h, m_v)
    mvec = m_v[...]

    row0 = sid * ROWS_PER_SUB
    pltpu.sync_copy(zeros_h.at[pl.ds(row0, ROWS_PER_SUB)],
                    acc_sh.at[pl.ds(row0, ROWS_PER_SUB)])
    plsc.subcore_barrier()

    lane = lax.iota(jnp.int32, L)

    def seg(si, _):
        off = wid * CHUNK + si * SEG
        pltpu.sync_copy(src_h.at[pl.ds(off, SEG)], src_v)
        pltpu.sync_copy(dst_h.at[pl.ds(off, SEG)], dst_v)

        for b in range(NBUF):
            svp = src_v[pl.ds(b * L, L)]
            pltpu.async_copy(haug_h.at[svp], gbufs[b], gsems[b])

        def use(g, b, issue_next, wait_scatter):
            sv = src_v[pl.ds(g * L, L)]
            dv = dst_v[pl.ds(g * L, L)]
            a = plsc.load_gather(ssrc_v, [sv])
            bsc = plsc.load_gather(sdst_v, [dv])
            t = a + bsc
            e = jnp.maximum(t, 0.2 * t)
            p = jnp.exp(e - mvec)
            gidx = off + g * L + lane
            p = jnp.where(gidx < E_TOT, p, 0.0)
            pltpu.make_async_copy(haug_h.at[sv], gbufs[b], gsems[b]).wait()
            for j in range(L):
                ps = p[j]
                for c in range(DA // L):
                    sl = pl.ds(c * L, L)
                    sbufs[b][j, sl] = gbufs[b][j, sl] * ps
            if issue_next is True:
                svn = src_v[pl.ds((g + NBUF) * L, L)]
                pltpu.async_copy(haug_h.at[svn], gbufs[b], gsems[b])
            elif issue_next is not False:
                @pl.when(issue_next)
                def _():
                    svn = src_v[pl.ds((g + NBUF) * L, L)]
                    pltpu.async_copy(haug_h.at[svn], gbufs[b], gsems[b])
            if wait_scatter:
                bp = (b - 1) % NBUF
                pltpu.make_async_copy(sbufs[bp], acc_sh.at[dv],
                                      ssems[0]).wait()
            pltpu.async_copy(sbufs[b], acc_sh.at[dv], ssems[0], add=True)

        for b in range(NBUF):
            use(b, b, NBUF + b < SBLK, b > 0)

        def group(gi, _):
            for b in range(NBUF):
                g = gi * NBUF + b
                use(g, b, g + NBUF < SBLK, True)
            return ()

        lax.fori_loop(1, SBLK // NBUF, group, (), unroll=False)
        for g in range((SBLK // NBUF) * NBUF, SBLK):
            use(g, g % NBUF, False, True)
        dv0 = dst_v[pl.ds(0, L)]
        pltpu.make_async_copy(sbufs[(SBLK - 1) % NBUF], acc_sh.at[dv0],
                              ssems[0]).wait()
        return ()

    lax.fori_loop(0, NSEG, seg, (), unroll=False)
    plsc.subcore_barrier()
    pltpu.sync_copy(acc_sh.at[pl.ds(row0, ROWS_PER_SUB)],
                    parts_h.at[cid].at[pl.ds(row0, ROWS_PER_SUB)])


_agg = functools.partial(
    pl.kernel,
    out_type=jax.ShapeDtypeStruct((NC, NPAD, DA), jnp.float32),
    mesh=_SC_MESH,
    compiler_params=pltpu.CompilerParams(needs_layout_passes=False, use_tc_tiling_on_sc=False),
    scratch_types=(
        [pltpu.VMEM((N,), jnp.float32),
         pltpu.VMEM((N,), jnp.float32),
         pltpu.VMEM((L,), jnp.float32),
         pltpu.VMEM((SEG,), jnp.int32),
         pltpu.VMEM((SEG,), jnp.int32)]
        + [pltpu.VMEM((L, DA), jnp.float32)] * (2 * NBUF)
        + [pltpu.VMEM_SHARED((NPAD, DA), jnp.float32)]
        + [pltpu.SemaphoreType.DMA] * (2 * NBUF)
    ),
)(_agg_body)


PPW = 2048 // NW
QPW = 256 // NW


def _gather_body(pa_h, pb_h, lin_h, b2_h, pids_h, xproj_h, prids_h,
                 pemb_h, xprojg_h,
                 pid_v, prid_v, ra_v, rb_v, linr_v, b2_v, out_v, xpr_v, sem):
    cid = lax.axis_index("c")
    sid = lax.axis_index("s")
    wid = cid * NS + sid

    pltpu.sync_copy(b2_h, b2_v)
    pltpu.sync_copy(pids_h.at[pl.ds(wid * PPW, PPW)], pid_v)
    pltpu.async_copy(pa_h.at[pid_v], ra_v, sem).wait()
    pltpu.async_copy(pb_h.at[pid_v], rb_v, sem).wait()
    pltpu.async_copy(lin_h.at[pid_v], linr_v, sem).wait()

    def row(j, _):
        dena = ra_v[j, pl.ds(D, DA - D)]
        denb = rb_v[j, pl.ds(D, DA - D)]
        invv = 1.0 / (dena + denb)
        inv = invv[0]
        for c in range(D // L):
            sl = pl.ds(c * L, L)
            out_v[j, sl] = ((ra_v[j, sl] + rb_v[j, sl]) * inv
                            + b2_v[pl.ds(c * L, L)] + linr_v[j, sl])
        return ()

    lax.fori_loop(0, PPW, row, (), unroll=False)
    pltpu.sync_copy(out_v, pemb_h.at[pl.ds(wid * PPW, PPW)])

    pltpu.sync_copy(prids_h.at[pl.ds(wid * QPW, QPW)], prid_v)
    pltpu.async_copy(xproj_h.at[prid_v], xpr_v, sem).wait()
    pltpu.sync_copy(xpr_v, xprojg_h.at[pl.ds(wid * QPW, QPW)])


def _gather_make(nproj, dproj):
    return functools.partial(
        pl.kernel,
        out_type=[
            jax.ShapeDtypeStruct((2048, D), jnp.float32),
            jax.ShapeDtypeStruct((256, dproj), jnp.float32),
        ],
        mesh=_SC_MESH,
        compiler_params=pltpu.CompilerParams(needs_layout_passes=False, use_tc_tiling_on_sc=False),
        scratch_types=[
            pltpu.VMEM((PPW,), jnp.int32),
            pltpu.VMEM((QPW,), jnp.int32),
            pltpu.VMEM((PPW, DA), jnp.float32),
            pltpu.VMEM((PPW, DA), jnp.float32),
            pltpu.VMEM((PPW, D), jnp.float32),
            pltpu.VMEM((D,), jnp.float32),
            pltpu.VMEM((PPW, D), jnp.float32),
            pltpu.VMEM((QPW, dproj), jnp.float32),
            pltpu.SemaphoreType.DMA,
        ],
    )(_gather_body)


def _head_body(xp_ref, wfc3_ref, bfc3_ref, wbot_ref, bfc1_ref,
               pe_ref, wtop_ref, wfc2_ref, bfc2_ref, out_ref):
    pr = jnp.dot(xp_ref[...], wfc3_ref[...],
                 preferred_element_type=jnp.float32) + bfc3_ref[...]
    pra = jnp.dot(pr, wbot_ref[...],
                  preferred_element_type=jnp.float32) + bfc1_ref[...]
    rep = jnp.broadcast_to(pra[:, None, :], (256, 8, D)).reshape(2048, D)
    mix = jnp.maximum(
        jnp.dot(pe_ref[...], wtop_ref[...], preferred_element_type=jnp.float32)
        + rep, 0.0)
    out_ref[...] = jnp.dot(mix, wfc2_ref[...],
                           preferred_element_type=jnp.float32) + bfc2_ref[...]


def kernel(x_person, edge_index, x_project, person_ids, project_ids, W1, a_src1, a_dst1, b1, Wl1, bl1, W2, a_src2, a_dst2, b2, Wl2, bl2, Wfc1, bfc1, Wfc2, bfc2, Wfc3, bfc3):
    loops = jnp.arange(N, dtype=edge_index.dtype)
    src = jnp.concatenate([edge_index[0], loops])
    dst = jnp.concatenate([edge_index[1], loops])
    pad = E_PAD - E_TOT
    src = jnp.pad(src, (0, pad))
    dst = jnp.pad(dst, (0, pad))
    zeros = jnp.zeros((NPAD, DA), jnp.float32)

    def mbound(mm):
        t = mm[0, 0] + mm[0, 1]
        return jnp.full((L,), jnp.maximum(t, 0.2 * t), jnp.float32)

    haug1, lin1, ss1, sd1, mm1 = _proj(x_person, W1, a_src1, a_dst1, Wl1, bl1)
    parts1 = _agg(src, dst, ss1[:, 0], sd1[:, 0], mbound(mm1), haug1, zeros)

    haug2, lin2, ss2, sd2, mm2 = _combine_proj(
        parts1[:, :N], lin1, b1, W2, a_src2, a_dst2, Wl2, bl2)
    parts2 = _agg(src, dst, ss2[:, 0], sd2[:, 0], mbound(mm2), haug2, zeros)

    pemb, xprojg = _gather_make(x_project.shape[0], x_project.shape[1])(
        parts2[0], parts2[1], lin2, b2, person_ids, x_project, project_ids)

    out = pl.pallas_call(
        _head_body,
        out_shape=jax.ShapeDtypeStruct((2048, 1), jnp.float32),
    )(xprojg, Wfc3, bfc3[None, :], Wfc1[D:], bfc1[None, :],
      pemb, Wfc1[:D], Wfc2, bfc2[None, :])
    return out.reshape(256, 2048 // 256)
